# Initial kernel scaffold; baseline (speedup 1.0000x reference)
import jax
import jax.numpy as jnp
from jax.experimental import pallas as pl


def kernel(x, edge_index, batch, W1, a_src1, a_dst1, b1, W2, a_src2, a_dst2, b2, Wl, bl):
    raise NotImplementedError("write your pallas kernel here")


# speedup vs baseline: 26.1109x; 26.1109x over previous
import functools

import jax
import jax.numpy as jnp
from jax import lax
from jax.experimental import pallas as pl
from jax.experimental.pallas import tpu as pltpu
from jax.experimental.pallas import tpu_sc as plsc

N = 10000
E = 320000
ETOT = E + N
DIN = 128
H = 3
C = 64
HC = H * C
HH = HC // 2
G = 64

NC = 2
NS = 16
GRP = 64
EPT = 20736
EPAD = NS * EPT
NGRP = EPT // GRP
RPT = N // NS

RB = 1000
NBLK = N // RB

f32 = jnp.float32
i32 = jnp.int32


def _feats_body(x_ref, w_ref, a_ref, hev_ref, hod_ref, t_ref, m_ref):
    i = pl.program_id(0)
    h = jnp.dot(x_ref[...], w_ref[...], preferred_element_type=f32)
    for t in range(6):
        hev_ref[:, t * 16:t * 16 + 16] = h[:, 32 * t:32 * t + 16]
        hod_ref[:, t * 16:t * 16 + 16] = h[:, 32 * t + 16:32 * t + 32]
    t8 = jnp.dot(h, a_ref[...], preferred_element_type=f32)
    t_ref[...] = t8
    mx = jnp.max(t8, axis=0, keepdims=True)

    @pl.when(i == 0)
    def _():
        m_ref[...] = mx

    @pl.when(i > 0)
    def _():
        m_ref[...] = jnp.maximum(m_ref[...], mx)


def _feats(x, W, A8):
    k = x.shape[1]
    return pl.pallas_call(
        _feats_body,
        grid=(NBLK,),
        in_specs=[
            pl.BlockSpec((RB, k), lambda i: (i, 0)),
            pl.BlockSpec((k, HC), lambda i: (0, 0)),
            pl.BlockSpec((HC, 8), lambda i: (0, 0)),
        ],
        out_specs=[
            pl.BlockSpec((RB, HH), lambda i: (i, 0)),
            pl.BlockSpec((RB, HH), lambda i: (i, 0)),
            pl.BlockSpec((RB, 8), lambda i: (i, 0)),
            pl.BlockSpec((1, 8), lambda i: (0, 0)),
        ],
        out_shape=[
            jax.ShapeDtypeStruct((N, HH), f32),
            jax.ShapeDtypeStruct((N, HH), f32),
            jax.ShapeDtypeStruct((N, 8), f32),
            jax.ShapeDtypeStruct((1, 8), f32),
        ],
    )(x, W, A8)


def _finish_body(acc_ref, den_ref, b_ref, o_ref):
    s = jnp.concatenate(
        [acc_ref[j % 2, :, (j // 2) * 16:(j // 2) * 16 + 16]
         for j in range(12)], axis=1)
    d = den_ref[0] + den_ref[1]
    for kk in range(H):
        sl = slice(64 * kk, 64 * kk + 64)
        r = 1.0 / d[:, kk:kk + 1]
        v = s[:, sl] * r + b_ref[0, sl][None, :]
        o_ref[:, sl] = jnp.where(v > 0.0, v, jnp.exp(v) - 1.0)


def _finish(acc, den, b):
    return pl.pallas_call(
        _finish_body,
        grid=(NBLK,),
        in_specs=[
            pl.BlockSpec((2, RB, HH), lambda i: (0, i, 0)),
            pl.BlockSpec((2, RB, 8), lambda i: (0, i, 0)),
            pl.BlockSpec((1, HC), lambda i: (0, 0)),
        ],
        out_specs=pl.BlockSpec((RB, HC), lambda i: (i, 0)),
        out_shape=jax.ShapeDtypeStruct((N, HC), f32),
    )(acc, den, b.reshape(1, HC))


def _pool_body(x_ref, b_ref, wl_ref, o_ref, accv, accc):
    i = pl.program_id(0)
    row = b_ref[0]
    gids = lax.broadcasted_iota(i32, (G, RB), 0)
    P = (gids == row).astype(f32)
    v = jnp.dot(x_ref[...], wl_ref[...], preferred_element_type=f32)

    @pl.when(i == 0)
    def _():
        accv[...] = jnp.zeros_like(accv)
        accc[...] = jnp.zeros_like(accc)

    accv[...] += jnp.dot(P, v, preferred_element_type=f32)
    accc[...] += jnp.dot(P, jnp.ones((RB, 128), f32),
                         preferred_element_type=f32)

    @pl.when(i == NBLK - 1)
    def _():
        o_ref[...] = accv[...] / jnp.maximum(accc[...], 1.0)


def _pool(x3, batch3, Wlp):
    return pl.pallas_call(
        _pool_body,
        grid=(NBLK,),
        in_specs=[
            pl.BlockSpec((RB, HC), lambda i: (i, 0)),
            pl.BlockSpec((1, 1, RB), lambda i: (i, 0, 0)),
            pl.BlockSpec((HC, 128), lambda i: (0, 0)),
        ],
        out_specs=pl.BlockSpec((G, 128), lambda i: (0, 0)),
        out_shape=jax.ShapeDtypeStruct((G, 128), f32),
        scratch_shapes=[pltpu.VMEM((G, 128), f32), pltpu.VMEM((G, 128), f32)],
    )(x3, batch3, Wlp)


def _edge_call(src3, dst3, tab8, hev, hod, z96, z8):
    mesh = plsc.VectorSubcoreMesh(core_axis_name="c", subcore_axis_name="s")

    @functools.partial(
        pl.kernel,
        out_type=[
            jax.ShapeDtypeStruct((NC, N, HH), f32),
            jax.ShapeDtypeStruct((NC, N, 8), f32),
        ],
        mesh=mesh,
        compiler_params=pltpu.CompilerParams(use_tc_tiling_on_sc=False,
                                             needs_layout_passes=False),
        scratch_types=[
            pltpu.VMEM((NGRP, GRP), i32),
            pltpu.VMEM((NGRP, GRP), i32),
            pltpu.VMEM((2, GRP, HH), f32),
            pltpu.VMEM((GRP, 8), f32),
            pltpu.VMEM((GRP, 8), f32),
            pltpu.VMEM((GRP, 8), f32),
            pltpu.VMEM_SHARED((N, HH), f32),
            pltpu.VMEM_SHARED((N, 8), f32),
            pltpu.SemaphoreType.DMA((2,)),
        ],
    )
    def edge_kernel(src_hbm, dst_hbm, tab_hbm, hev_hbm, hod_hbm,
                    z96_hbm, z8_hbm, acc_out, den_out,
                    srcb, dstb, rows, tsrc, tdst, exb,
                    acc, dacc, gsem):
        c = lax.axis_index("c")
        s = lax.axis_index("s")
        rbase = s * RPT
        ebase = s * EPT

        pltpu.sync_copy(src_hbm.at[s], srcb)
        pltpu.sync_copy(dst_hbm.at[s], dstb)
        pltpu.sync_copy(z96_hbm, acc.at[pl.ds(rbase, RPT)])
        pltpu.sync_copy(z8_hbm, dacc.at[pl.ds(rbase, RPT)])

        zero16 = jnp.zeros((16,), f32)
        lanes0 = lax.iota(i32, 16)
        for ii in range(4):
            for kk in range(3, 8):
                plsc.store_scatter(
                    exb, [lanes0 + ii * 16, jnp.full((16,), kk, i32)], zero16)

        plsc.subcore_barrier()

        def gather_start(g, b):
            @pl.when(c == 0)
            def _():
                pltpu.async_copy(hev_hbm.at[srcb.at[g]], rows.at[b],
                                 gsem.at[b])

            @pl.when(c == 1)
            def _():
                pltpu.async_copy(hod_hbm.at[srcb.at[g]], rows.at[b],
                                 gsem.at[b])

        def gather_wait(g, b):
            @pl.when(c == 0)
            def _():
                pltpu.make_async_copy(
                    hev_hbm.at[srcb.at[g]], rows.at[b], gsem.at[b]).wait()

            @pl.when(c == 1)
            def _():
                pltpu.make_async_copy(
                    hod_hbm.at[srcb.at[g]], rows.at[b], gsem.at[b]).wait()

        def do_group(g, b):
            pltpu.sync_copy(tab_hbm.at[srcb.at[g]], tsrc)
            pltpu.sync_copy(tab_hbm.at[dstb.at[g]], tdst)
            for ii in range(GRP // 16):
                lanes = lanes0 + ii * 16
                valid = (lanes + (ebase + g * GRP)) < ETOT
                mv = plsc.load_gather(tsrc, [lanes, jnp.full((16,), 6, i32)])
                for kk in range(H):
                    es = plsc.load_gather(
                        tsrc, [lanes, jnp.full((16,), kk, i32)])
                    ed = plsc.load_gather(
                        tdst, [lanes, jnp.full((16,), 3 + kk, i32)])
                    e = es + ed
                    e = jnp.maximum(e, 0.2 * e)
                    ex = jnp.exp(e - mv)
                    ex = jnp.where(valid, ex, 0.0)
                    plsc.store_scatter(
                        exb, [lanes, jnp.full((16,), kk, i32)], ex)
            gather_wait(g, b)

            def scale_body(e2, carry):
                e2v = jnp.full((16,), 0, i32) + e2
                for kk in range(H):
                    av = plsc.load_gather(exb, [e2v, jnp.full((16,), kk, i32)])
                    for jj in range(2):
                        sl = pl.ds(kk * 32 + jj * 16, 16)
                        rows[b, e2, sl] = rows[b, e2, sl] * av
                return carry

            lax.fori_loop(0, GRP, scale_body, 0)
            pltpu.sync_copy(rows.at[b], acc.at[dstb.at[g]], add=True)

            @pl.when(c == 0)
            def _():
                pltpu.sync_copy(exb, dacc.at[dstb.at[g]], add=True)

        gather_start(0, 0)

        def pair_body(g2, carry):
            for b in (0, 1):
                g = g2 * 2 + b

                @pl.when(g + 1 < NGRP)
                def _():
                    gather_start(g + 1, 1 - b)

                do_group(g, b)
            return carry

        lax.fori_loop(0, NGRP // 2, pair_body, 0)
        plsc.subcore_barrier()

        pltpu.sync_copy(acc.at[pl.ds(rbase, RPT)],
                        acc_out.at[c, pl.ds(rbase, RPT)])
        pltpu.sync_copy(dacc.at[pl.ds(rbase, RPT)],
                        den_out.at[c, pl.ds(rbase, RPT)])

    return edge_kernel(src3, dst3, tab8, hev, hod, z96, z8)


def _build_A(a_src, a_dst):
    rows = jnp.arange(HC)
    head = rows // C
    A = jnp.zeros((HC, 8), f32)
    A = A.at[rows, head].set(a_src.reshape(HC))
    A = A.at[rows, head + 3].set(a_dst.reshape(HC))
    return A


def _make_tab(t8, m8):
    M = jnp.maximum(jnp.max(m8[0, :3]) + jnp.max(m8[0, 3:6]), 0.0)
    return jnp.concatenate(
        [t8[:, :6], jnp.full((N, 1), M, f32), jnp.zeros((N, 1), f32)], axis=1)


def kernel(x, edge_index, batch, W1, a_src1, a_dst1, b1,
           W2, a_src2, a_dst2, b2, Wl, bl):
    loops = jnp.arange(N, dtype=i32)
    pad = jnp.zeros((EPAD - ETOT,), i32)
    src3 = jnp.concatenate([edge_index[0], loops, pad]).reshape(NS, NGRP, GRP)
    dst3 = jnp.concatenate([edge_index[1], loops, pad]).reshape(NS, NGRP, GRP)
    z96 = jnp.zeros((RPT, HH), f32)
    z8 = jnp.zeros((RPT, 8), f32)
    A1 = _build_A(a_src1, a_dst1)
    A2 = _build_A(a_src2, a_dst2)
    Wlp = jnp.zeros((HC, 128), f32).at[:, 0].set(Wl[:, 0])

    hev1, hod1, t8_1, m8_1 = _feats(x, W1, A1)
    acc1, den1 = _edge_call(src3, dst3, _make_tab(t8_1, m8_1),
                            hev1, hod1, z96, z8)
    x2 = _finish(acc1, den1, b1)

    hev2, hod2, t8_2, m8_2 = _feats(x2, W2, A2)
    acc2, den2 = _edge_call(src3, dst3, _make_tab(t8_2, m8_2),
                            hev2, hod2, z96, z8)
    x3 = _finish(acc2, den2, b2)

    pooled = _pool(x3, batch.reshape(NBLK, 1, RB), Wlp)
    return pooled[:, 0] + bl[0]
